# baseline (speedup 1.0000x reference)
import jax
import jax.numpy as jnp
from jax.experimental import pallas as pl

_OFFS = jnp.array(
    [[0, 0, 0], [1, 0, 0], [0, 1, 0], [0, 0, 1],
     [1, 1, 0], [0, 1, 1], [1, 0, 1], [1, 1, 1]], dtype=jnp.int32)

_R = 1000


def _body(offs_ref, inds_ref, feats_ref, oinds_ref, ofeats_ref):
    inds = inds_ref[...]
    feats = feats_ref[...]
    oinds_ref[...] = inds[:, None, :] * 2 + offs_ref[...][None, :, :]
    ofeats_ref[...] = jnp.broadcast_to(feats[:, None, :], (_R, 8, 64))


def kernel(voxel_inds, feats):
    n = feats.shape[0]
    grid = n // _R
    oinds3, ofeats3 = pl.pallas_call(
        _body,
        grid=(grid,),
        in_specs=[
            pl.BlockSpec((8, 3), lambda i: (0, 0)),
            pl.BlockSpec((_R, 3), lambda i: (i, 0)),
            pl.BlockSpec((_R, 64), lambda i: (i, 0)),
        ],
        out_specs=[
            pl.BlockSpec((_R, 8, 3), lambda i: (i, 0, 0)),
            pl.BlockSpec((_R, 8, 64), lambda i: (i, 0, 0)),
        ],
        out_shape=[
            jax.ShapeDtypeStruct((n, 8, 3), jnp.int32),
            jax.ShapeDtypeStruct((n, 8, 64), jnp.float32),
        ],
    )(_OFFS, voxel_inds, feats)
    return oinds3.reshape(n * 8, 3), ofeats3.reshape(n * 8, 64)
